# baseline (speedup 1.0000x reference)
import functools

import jax
import jax.numpy as jnp
from jax import lax
from jax.experimental import pallas as pl
from jax.experimental.pallas import tpu as pltpu
from jax.experimental.pallas import tpu_sc as plsc

N = 10000
D_IN = 128
DIM = 16
N_CLASSES = 16
E = 320000

NC = 2
NS = 16
NT = NC * NS
CH = 125
CHUNKS = 80
NBUF = 8
NOUT = CHUNKS // NBUF
NP = N
RPT = 632
RPT_LAST = N - (NS - 1) * RPT


def _over_rows(s, fn):
  @pl.when(s < NS - 1)
  def _():
    fn(pl.ds(s * RPT, RPT))

  @pl.when(s == NS - 1)
  def _():
    fn(pl.ds((NS - 1) * RPT, RPT_LAST))

RPT16 = 640
RPT16_LAST = N - (NS - 1) * RPT16


@functools.lru_cache(maxsize=None)
def _make_sc_deg():
  mesh = plsc.VectorSubcoreMesh(core_axis_name="c", subcore_axis_name="s",
                                num_cores=NC, num_subcores=NS)
  EPT = CHUNKS * CH

  scratch = [
      pltpu.VMEM((EPT,), jnp.int32),
      pltpu.VMEM((N,), jnp.float32),
      pltpu.VMEM((NS, RPT16), jnp.float32),
      pltpu.VMEM((RPT16,), jnp.float32),
      pltpu.VMEM_SHARED((NS, N), jnp.float32),
  ]

  def body(dstf_hbm, zeros1_hbm, out0_hbm, out1_hbm,
           dst_f, hist, tmp_v, red_v, hist_sh):
    c = lax.axis_index("c")
    s = lax.axis_index("s")
    wid = s * NC + c
    pltpu.sync_copy(dstf_hbm.at[wid], dst_f)
    pltpu.sync_copy(zeros1_hbm, hist)
    ones16 = jnp.full((16,), 1.0, jnp.float32)

    def step(i, _):
      idx = dst_f[pl.ds(i * 16, 16)]
      plsc.addupdate_scatter(hist, [idx], ones16)
      return 0

    lax.fori_loop(0, EPT // 16, step, 0)
    pltpu.sync_copy(hist, hist_sh.at[s])
    plsc.subcore_barrier()

    def reduce_range(base, length):
      for t in range(NS):
        pltpu.sync_copy(hist_sh.at[t, pl.ds(base, length)],
                        tmp_v.at[t, pl.ds(0, length)])

      def rstep(b2, _):
        acc = tmp_v[0, pl.ds(b2 * 16, 16)]
        for t in range(1, NS):
          acc = acc + tmp_v[t, pl.ds(b2 * 16, 16)]
        red_v[pl.ds(b2 * 16, 16)] = acc
        return 0

      lax.fori_loop(0, length // 16, rstep, 0)

      @pl.when(c == 0)
      def _():
        pltpu.sync_copy(red_v.at[pl.ds(0, length)],
                        out0_hbm.at[pl.ds(base, length)])

      @pl.when(c != 0)
      def _():
        pltpu.sync_copy(red_v.at[pl.ds(0, length)],
                        out1_hbm.at[pl.ds(base, length)])

    @pl.when(s < NS - 1)
    def _():
      reduce_range(s * RPT16, RPT16)

    @pl.when(s == NS - 1)
    def _():
      reduce_range((NS - 1) * RPT16, RPT16_LAST)

  return pl.kernel(
      body,
      out_type=[
          jax.ShapeDtypeStruct((N,), jnp.float32),
          jax.ShapeDtypeStruct((N,), jnp.float32),
      ],
      mesh=mesh,
      scratch_types=scratch,
      compiler_params=pltpu.CompilerParams(use_tc_tiling_on_sc=False,
                                           needs_layout_passes=False),
  )


@functools.lru_cache(maxsize=None)
def _make_sc_agg():
  mesh = plsc.VectorSubcoreMesh(core_axis_name="c", subcore_axis_name="s",
                                num_cores=NC, num_subcores=NS)

  scratch = [
      pltpu.VMEM((CHUNKS, CH), jnp.int32),
      pltpu.VMEM((CHUNKS, CH), jnp.int32),
      pltpu.VMEM((NBUF, CH, DIM), jnp.float32),
      pltpu.VMEM_SHARED((NP, DIM), jnp.float32),
      pltpu.VMEM_SHARED((N, DIM), jnp.float32),
      pltpu.SemaphoreType.DMA((NBUF,)),
  ]

  def body(y_hbm, src_hbm, dst_hbm, zeros_hbm, out0_hbm, out1_hbm,
           src_v, dst_v, rows_v, z_sh, y_sh, sem):
    c = lax.axis_index("c")
    s = lax.axis_index("s")
    wid = s * NC + c

    _over_rows(s, lambda r: pltpu.sync_copy(zeros_hbm.at[r], z_sh.at[r]))
    _over_rows(s, lambda r: pltpu.sync_copy(y_hbm.at[r], y_sh.at[r]))

    pltpu.sync_copy(src_hbm.at[wid], src_v)
    pltpu.sync_copy(dst_hbm.at[wid], dst_v)
    plsc.subcore_barrier()

    def start_gather(b, j):
      from_hbm = (j % 5) < 2

      @pl.when(from_hbm)
      def _():
        pltpu.async_copy(y_hbm.at[src_v.at[j]], rows_v.at[b], sem.at[b])

      @pl.when(jnp.logical_not(from_hbm))
      def _():
        pltpu.async_copy(y_sh.at[src_v.at[j]], rows_v.at[b], sem.at[b])

    for b in range(NBUF):
      start_gather(b, b)

    def outer(o, _):
      for b in range(NBUF):
        j = o * NBUF + b
        pltpu.make_async_copy(
            y_sh.at[src_v.at[j]], rows_v.at[b], sem.at[b]).wait()
        pltpu.sync_copy(rows_v.at[b], z_sh.at[dst_v.at[j]], add=True)
        nxt = j + NBUF

        @pl.when(nxt < CHUNKS)
        def _():
          start_gather(b, nxt)
      return 0

    lax.fori_loop(0, NOUT, outer, 0)
    plsc.subcore_barrier()

    @pl.when(c == 0)
    def _():
      _over_rows(s, lambda r: pltpu.sync_copy(z_sh.at[r], out0_hbm.at[r]))

    @pl.when(c != 0)
    def _():
      _over_rows(s, lambda r: pltpu.sync_copy(z_sh.at[r], out1_hbm.at[r]))

  return pl.kernel(
      body,
      out_type=[
          jax.ShapeDtypeStruct((N, DIM), jnp.float32),
          jax.ShapeDtypeStruct((N, DIM), jnp.float32),
      ],
      mesh=mesh,
      scratch_types=scratch,
      compiler_params=pltpu.CompilerParams(use_tc_tiling_on_sc=False),
  )


P = 8
NB = N // P
LANES = P * DIM


def _blockdiag(w):
  k, m = w.shape
  return (jnp.eye(P, dtype=w.dtype)[:, None, :, None]
          * w[None, :, None, :]).reshape(P * k, P * m)


def _tc_mm1_body(x8_ref, w_ref, xt_ref):
  xt_ref[...] = jnp.dot(x8_ref[...], w_ref[...],
                        preferred_element_type=jnp.float32)


_tc_mm1 = pl.pallas_call(
    _tc_mm1_body,
    out_shape=jax.ShapeDtypeStruct((NB, LANES), jnp.float32),
)


def _tc_prep1_body(d0_ref, d1_ref, bmat_ref, xt_ref, y_ref, dinv_ref):
  deg8 = d0_ref[...] + d1_ref[...] + 1.0
  dinv = jnp.dot(lax.rsqrt(deg8), bmat_ref[...],
                 preferred_element_type=jnp.float32)
  dinv_ref[...] = dinv
  y_ref[...] = dinv * xt_ref[...]


_tc_prep1 = pl.pallas_call(
    _tc_prep1_body,
    out_shape=[
        jax.ShapeDtypeStruct((NB, LANES), jnp.float32),
        jax.ShapeDtypeStruct((NB, LANES), jnp.float32),
    ],
)


def _tc_mid_body(z0_ref, z1_ref, yin_ref, dinv_ref, b_ref, w_ref, y_ref):
  dinv = dinv_ref[...]
  h = jax.nn.relu(dinv * (z0_ref[...] + z1_ref[...] + yin_ref[...])
                  + b_ref[...])
  y_ref[...] = dinv * jnp.dot(h, w_ref[...],
                              preferred_element_type=jnp.float32)


_tc_mid = pl.pallas_call(
    _tc_mid_body,
    out_shape=jax.ShapeDtypeStruct((NB, LANES), jnp.float32),
)


def _tc_final_body(z0_ref, z1_ref, yin_ref, dinv_ref, b_ref, wl_ref, bl_ref,
                   g_ref, o_ref):
  h = dinv_ref[...] * (z0_ref[...] + z1_ref[...] + yin_ref[...]) + b_ref[...]
  lg = jnp.dot(h, wl_ref[...], preferred_element_type=jnp.float32)
  lg = lg + bl_ref[...]
  m = jnp.max(lg, axis=1, keepdims=True)
  ex = jnp.exp(lg - m)
  s = jnp.dot(ex, g_ref[...], preferred_element_type=jnp.float32)
  o_ref[...] = (lg - m) - jnp.log(s)


_tc_final = pl.pallas_call(
    _tc_final_body,
    out_shape=jax.ShapeDtypeStruct((NB, LANES), jnp.float32),
)


def kernel(x, edge_index, W1, b1, W2, b2, W3, b3, Wl, bl):
  src_p = edge_index[0].reshape(NT, CHUNKS, CH)
  dst_p = edge_index[1].reshape(NT, CHUNKS, CH)
  dst_f = edge_index[1].reshape(NT, CHUNKS * CH)

  zeros = jnp.zeros((N, DIM), jnp.float32)
  zeros1 = jnp.zeros((N,), jnp.float32)

  sc_layer = _make_sc_agg()
  sc_deg = _make_sc_deg()

  x8 = x.reshape(NB, P * D_IN)
  w1b = _blockdiag(W1)
  w2b = _blockdiag(W2)
  w3b = _blockdiag(W3)
  wlb = _blockdiag(Wl)
  gmat = _blockdiag(jnp.ones((DIM, N_CLASSES), jnp.float32))
  bmat = _blockdiag(jnp.ones((1, DIM), jnp.float32))
  b1p = jnp.tile(b1, P).reshape(1, LANES)
  b2p = jnp.tile(b2, P).reshape(1, LANES)
  b3p = jnp.tile(b3, P).reshape(1, LANES)
  blp = jnp.tile(bl, P).reshape(1, LANES)

  pk = lambda a: a.reshape(NB, LANES)

  xt1 = _tc_mm1(x8, w1b)
  d0, d1 = sc_deg(dst_f, zeros1)
  y1, dinv = _tc_prep1(d0.reshape(NB, P), d1.reshape(NB, P), bmat, xt1)
  z0, z1 = sc_layer(y1.reshape(N, DIM), src_p, dst_p, zeros)
  y2 = _tc_mid(pk(z0), pk(z1), y1, dinv, b1p, w2b)
  z0, z1 = sc_layer(y2.reshape(N, DIM), src_p, dst_p, zeros)
  y3 = _tc_mid(pk(z0), pk(z1), y2, dinv, b2p, w3b)
  z0, z1 = sc_layer(y3.reshape(N, DIM), src_p, dst_p, zeros)
  out = _tc_final(pk(z0), pk(z1), y3, dinv, b3p, wlb, blp, gmat)
  return out.reshape(N, N_CLASSES)


# speedup vs baseline: 1.0328x; 1.0328x over previous
import functools

import jax
import jax.numpy as jnp
from jax import lax
from jax.experimental import pallas as pl
from jax.experimental.pallas import tpu as pltpu
from jax.experimental.pallas import tpu_sc as plsc

N = 10000
D_IN = 128
DIM = 16
N_CLASSES = 16
E = 320000

NC = 2
NS = 16
NT = NC * NS
CH = 125
CHUNKS = 80
NBUF = 8
NOUT = CHUNKS // NBUF
NP = N
RPT = 632
RPT_LAST = N - (NS - 1) * RPT


def _over_rows(s, fn):
  @pl.when(s < NS - 1)
  def _():
    fn(pl.ds(s * RPT, RPT))

  @pl.when(s == NS - 1)
  def _():
    fn(pl.ds((NS - 1) * RPT, RPT_LAST))

RPT16 = 640
RPT16_LAST = N - (NS - 1) * RPT16


@functools.lru_cache(maxsize=None)
def _make_sc_deg():
  mesh = plsc.VectorSubcoreMesh(core_axis_name="c", subcore_axis_name="s",
                                num_cores=NC, num_subcores=NS)
  EPT = CHUNKS * CH

  scratch = [
      pltpu.VMEM((EPT,), jnp.int32),
      pltpu.VMEM((N,), jnp.float32),
      pltpu.VMEM((NS, RPT16), jnp.float32),
      pltpu.VMEM((RPT16, DIM), jnp.float32),
      pltpu.VMEM_SHARED((NS, N), jnp.float32),
  ]

  def body(dstf_hbm, zeros1_hbm, out0_hbm, out1_hbm,
           dst_f, hist, tmp_v, rep_v, hist_sh):
    c = lax.axis_index("c")
    s = lax.axis_index("s")
    wid = s * NC + c
    pltpu.sync_copy(dstf_hbm.at[wid], dst_f)
    pltpu.sync_copy(zeros1_hbm, hist)
    ones16 = jnp.full((16,), 1.0, jnp.float32)

    def step(i, _):
      idx = dst_f[pl.ds(i * 16, 16)]
      plsc.addupdate_scatter(hist, [idx], ones16)
      return 0

    lax.fori_loop(0, EPT // 16, step, 0)
    pltpu.sync_copy(hist, hist_sh.at[s])
    plsc.subcore_barrier()

    def reduce_range(base, length):
      for t in range(NS):
        pltpu.sync_copy(hist_sh.at[t, pl.ds(base, length)],
                        tmp_v.at[t, pl.ds(0, length)])

      def rstep(b2, _):
        acc = tmp_v[0, pl.ds(b2 * 16, 16)]
        for t in range(1, NS):
          acc = acc + tmp_v[t, pl.ds(b2 * 16, 16)]
        for k in range(16):
          rep_v[b2 * 16 + k, :] = jnp.full((DIM,), acc[k], jnp.float32)
        return 0

      lax.fori_loop(0, length // 16, rstep, 0)

      @pl.when(c == 0)
      def _():
        pltpu.sync_copy(rep_v.at[pl.ds(0, length)],
                        out0_hbm.at[pl.ds(base, length)])

      @pl.when(c != 0)
      def _():
        pltpu.sync_copy(rep_v.at[pl.ds(0, length)],
                        out1_hbm.at[pl.ds(base, length)])

    @pl.when(s < NS - 1)
    def _():
      reduce_range(s * RPT16, RPT16)

    @pl.when(s == NS - 1)
    def _():
      reduce_range((NS - 1) * RPT16, RPT16_LAST)

  return pl.kernel(
      body,
      out_type=[
          jax.ShapeDtypeStruct((N, DIM), jnp.float32),
          jax.ShapeDtypeStruct((N, DIM), jnp.float32),
      ],
      mesh=mesh,
      scratch_types=scratch,
      compiler_params=pltpu.CompilerParams(use_tc_tiling_on_sc=False,
                                           needs_layout_passes=False),
  )


@functools.lru_cache(maxsize=None)
def _make_sc_agg():
  mesh = plsc.VectorSubcoreMesh(core_axis_name="c", subcore_axis_name="s",
                                num_cores=NC, num_subcores=NS)

  scratch = [
      pltpu.VMEM((CHUNKS, CH), jnp.int32),
      pltpu.VMEM((CHUNKS, CH), jnp.int32),
      pltpu.VMEM((NBUF, CH, DIM), jnp.float32),
      pltpu.VMEM_SHARED((NP, DIM), jnp.float32),
      pltpu.VMEM_SHARED((N, DIM), jnp.float32),
      pltpu.SemaphoreType.DMA((NBUF,)),
  ]

  def body(y_hbm, src_hbm, dst_hbm, zeros_hbm, out0_hbm, out1_hbm,
           src_v, dst_v, rows_v, z_sh, y_sh, sem):
    c = lax.axis_index("c")
    s = lax.axis_index("s")
    wid = s * NC + c

    _over_rows(s, lambda r: pltpu.sync_copy(zeros_hbm.at[r], z_sh.at[r]))
    _over_rows(s, lambda r: pltpu.sync_copy(y_hbm.at[r], y_sh.at[r]))

    pltpu.sync_copy(src_hbm.at[wid], src_v)
    pltpu.sync_copy(dst_hbm.at[wid], dst_v)
    plsc.subcore_barrier()

    def start_gather(b, j):
      from_hbm = (j % 5) < 2

      @pl.when(from_hbm)
      def _():
        pltpu.async_copy(y_hbm.at[src_v.at[j]], rows_v.at[b], sem.at[b])

      @pl.when(jnp.logical_not(from_hbm))
      def _():
        pltpu.async_copy(y_sh.at[src_v.at[j]], rows_v.at[b], sem.at[b])

    for b in range(NBUF):
      start_gather(b, b)

    def outer(o, _):
      for b in range(NBUF):
        j = o * NBUF + b
        pltpu.make_async_copy(
            y_sh.at[src_v.at[j]], rows_v.at[b], sem.at[b]).wait()
        pltpu.sync_copy(rows_v.at[b], z_sh.at[dst_v.at[j]], add=True)
        nxt = j + NBUF

        @pl.when(nxt < CHUNKS)
        def _():
          start_gather(b, nxt)
      return 0

    lax.fori_loop(0, NOUT, outer, 0)
    plsc.subcore_barrier()

    @pl.when(c == 0)
    def _():
      _over_rows(s, lambda r: pltpu.sync_copy(z_sh.at[r], out0_hbm.at[r]))

    @pl.when(c != 0)
    def _():
      _over_rows(s, lambda r: pltpu.sync_copy(z_sh.at[r], out1_hbm.at[r]))

  return pl.kernel(
      body,
      out_type=[
          jax.ShapeDtypeStruct((N, DIM), jnp.float32),
          jax.ShapeDtypeStruct((N, DIM), jnp.float32),
      ],
      mesh=mesh,
      scratch_types=scratch,
      compiler_params=pltpu.CompilerParams(use_tc_tiling_on_sc=False),
  )


P = 8
NB = N // P
LANES = P * DIM


def _blockdiag(w):
  k, m = w.shape
  return (jnp.eye(P, dtype=w.dtype)[:, None, :, None]
          * w[None, :, None, :]).reshape(P * k, P * m)


def _tc_mm1_body(x8_ref, w_ref, xt_ref):
  xt_ref[...] = jnp.dot(x8_ref[...], w_ref[...],
                        preferred_element_type=jnp.float32)


_tc_mm1 = pl.pallas_call(
    _tc_mm1_body,
    out_shape=jax.ShapeDtypeStruct((NB, LANES), jnp.float32),
)


def _tc_prep1_body(d0_ref, d1_ref, xt_ref, y_ref, dinv_ref):
  deg = d0_ref[...] + d1_ref[...] + 1.0
  dinv = lax.rsqrt(deg)
  dinv_ref[...] = dinv
  y_ref[...] = dinv * xt_ref[...]


_tc_prep1 = pl.pallas_call(
    _tc_prep1_body,
    out_shape=[
        jax.ShapeDtypeStruct((NB, LANES), jnp.float32),
        jax.ShapeDtypeStruct((NB, LANES), jnp.float32),
    ],
)


def _tc_mid_body(z0_ref, z1_ref, yin_ref, dinv_ref, b_ref, w_ref, y_ref):
  dinv = dinv_ref[...]
  h = jax.nn.relu(dinv * (z0_ref[...] + z1_ref[...] + yin_ref[...])
                  + b_ref[...])
  y_ref[...] = dinv * jnp.dot(h, w_ref[...],
                              preferred_element_type=jnp.float32)


_tc_mid = pl.pallas_call(
    _tc_mid_body,
    out_shape=jax.ShapeDtypeStruct((NB, LANES), jnp.float32),
)


def _tc_final_body(z0_ref, z1_ref, yin_ref, dinv_ref, b_ref, wl_ref, bl_ref,
                   g_ref, o_ref):
  h = dinv_ref[...] * (z0_ref[...] + z1_ref[...] + yin_ref[...]) + b_ref[...]
  lg = jnp.dot(h, wl_ref[...], preferred_element_type=jnp.float32)
  lg = lg + bl_ref[...]
  m = jnp.max(lg, axis=1, keepdims=True)
  ex = jnp.exp(lg - m)
  s = jnp.dot(ex, g_ref[...], preferred_element_type=jnp.float32)
  o_ref[...] = (lg - m) - jnp.log(s)


_tc_final = pl.pallas_call(
    _tc_final_body,
    out_shape=jax.ShapeDtypeStruct((NB, LANES), jnp.float32),
)


def kernel(x, edge_index, W1, b1, W2, b2, W3, b3, Wl, bl):
  src_p = edge_index[0].reshape(NT, CHUNKS, CH)
  dst_p = edge_index[1].reshape(NT, CHUNKS, CH)
  dst_f = edge_index[1].reshape(NT, CHUNKS * CH)

  zeros = jnp.zeros((N, DIM), jnp.float32)
  zeros1 = jnp.zeros((N,), jnp.float32)

  sc_layer = _make_sc_agg()
  sc_deg = _make_sc_deg()

  x8 = x.reshape(NB, P * D_IN)
  w1b = _blockdiag(W1)
  w2b = _blockdiag(W2)
  w3b = _blockdiag(W3)
  wlb = _blockdiag(Wl)
  gmat = _blockdiag(jnp.ones((DIM, N_CLASSES), jnp.float32))
  b1p = jnp.tile(b1, P).reshape(1, LANES)
  b2p = jnp.tile(b2, P).reshape(1, LANES)
  b3p = jnp.tile(b3, P).reshape(1, LANES)
  blp = jnp.tile(bl, P).reshape(1, LANES)

  pk = lambda a: a.reshape(NB, LANES)

  xt1 = _tc_mm1(x8, w1b)
  d0, d1 = sc_deg(dst_f, zeros1)
  y1, dinv = _tc_prep1(pk(d0), pk(d1), xt1)
  z0, z1 = sc_layer(y1.reshape(N, DIM), src_p, dst_p, zeros)
  y2 = _tc_mid(pk(z0), pk(z1), y1, dinv, b1p, w2b)
  z0, z1 = sc_layer(y2.reshape(N, DIM), src_p, dst_p, zeros)
  y3 = _tc_mid(pk(z0), pk(z1), y2, dinv, b2p, w3b)
  z0, z1 = sc_layer(y3.reshape(N, DIM), src_p, dst_p, zeros)
  out = _tc_final(pk(z0), pk(z1), y3, dinv, b3p, wlb, blp, gmat)
  return out.reshape(N, N_CLASSES)


# speedup vs baseline: 1.0462x; 1.0130x over previous
import functools

import jax
import jax.numpy as jnp
from jax import lax
from jax.experimental import pallas as pl
from jax.experimental.pallas import tpu as pltpu
from jax.experimental.pallas import tpu_sc as plsc

N = 10000
D_IN = 128
DIM = 16
N_CLASSES = 16
E = 320000

NC = 2
NS = 16
NT = NC * NS
CH = 125
CHUNKS = 80
NBUF = 8
NOUT = CHUNKS // NBUF
NP = N
RPT = 632
RPT_LAST = N - (NS - 1) * RPT


def _over_rows(s, fn):
  @pl.when(s < NS - 1)
  def _():
    fn(pl.ds(s * RPT, RPT))

  @pl.when(s == NS - 1)
  def _():
    fn(pl.ds((NS - 1) * RPT, RPT_LAST))

RPT16 = 640
RPT16_LAST = N - (NS - 1) * RPT16


@functools.lru_cache(maxsize=None)
def _make_sc_deg():
  mesh = plsc.VectorSubcoreMesh(core_axis_name="c", subcore_axis_name="s",
                                num_cores=NC, num_subcores=NS)
  EPT = CHUNKS * CH

  scratch = [
      pltpu.VMEM((EPT,), jnp.int32),
      pltpu.VMEM((N,), jnp.float32),
      pltpu.VMEM((NS, RPT16), jnp.float32),
      pltpu.VMEM((RPT16, DIM), jnp.float32),
      pltpu.VMEM_SHARED((NS, N), jnp.float32),
  ]

  def body(dstf_hbm, zeros1_hbm, out0_hbm, out1_hbm,
           dst_f, hist, tmp_v, rep_v, hist_sh):
    c = lax.axis_index("c")
    s = lax.axis_index("s")
    wid = s * NC + c
    pltpu.sync_copy(dstf_hbm.at[wid], dst_f)
    pltpu.sync_copy(zeros1_hbm, hist)
    ones16 = jnp.full((16,), 1.0, jnp.float32)

    def step(i, _):
      idx = dst_f[pl.ds(i * 16, 16)]
      plsc.addupdate_scatter(hist, [idx], ones16)
      return 0

    lax.fori_loop(0, EPT // 16, step, 0)
    pltpu.sync_copy(hist, hist_sh.at[s])
    plsc.subcore_barrier()

    def reduce_range(base, length):
      for t in range(NS):
        pltpu.sync_copy(hist_sh.at[t, pl.ds(base, length)],
                        tmp_v.at[t, pl.ds(0, length)])

      def rstep(b2, _):
        acc = tmp_v[0, pl.ds(b2 * 16, 16)]
        for t in range(1, NS):
          acc = acc + tmp_v[t, pl.ds(b2 * 16, 16)]
        for k in range(16):
          rep_v[b2 * 16 + k, :] = jnp.full((DIM,), acc[k], jnp.float32)
        return 0

      lax.fori_loop(0, length // 16, rstep, 0)

      @pl.when(c == 0)
      def _():
        pltpu.sync_copy(rep_v.at[pl.ds(0, length)],
                        out0_hbm.at[pl.ds(base, length)])

      @pl.when(c != 0)
      def _():
        pltpu.sync_copy(rep_v.at[pl.ds(0, length)],
                        out1_hbm.at[pl.ds(base, length)])

    @pl.when(s < NS - 1)
    def _():
      reduce_range(s * RPT16, RPT16)

    @pl.when(s == NS - 1)
    def _():
      reduce_range((NS - 1) * RPT16, RPT16_LAST)

  return pl.kernel(
      body,
      out_type=[
          jax.ShapeDtypeStruct((N, DIM), jnp.float32),
          jax.ShapeDtypeStruct((N, DIM), jnp.float32),
      ],
      mesh=mesh,
      scratch_types=scratch,
      compiler_params=pltpu.CompilerParams(use_tc_tiling_on_sc=False,
                                           needs_layout_passes=False),
  )


@functools.lru_cache(maxsize=None)
def _make_sc_agg():
  mesh = plsc.VectorSubcoreMesh(core_axis_name="c", subcore_axis_name="s",
                                num_cores=NC, num_subcores=NS)

  scratch = [
      pltpu.VMEM((CHUNKS, CH), jnp.int32),
      pltpu.VMEM((CHUNKS, CH), jnp.int32),
      pltpu.VMEM((NBUF, CH, DIM), jnp.float32),
      pltpu.VMEM_SHARED((NP, DIM), jnp.float32),
      pltpu.VMEM_SHARED((N, DIM), jnp.float32),
      pltpu.SemaphoreType.DMA((NBUF,)),
  ]

  def body(y_hbm, src_hbm, dst_hbm, zeros_hbm, out0_hbm, out1_hbm,
           src_v, dst_v, rows_v, z_sh, y_sh, sem):
    c = lax.axis_index("c")
    s = lax.axis_index("s")
    wid = s * NC + c

    _over_rows(s, lambda r: pltpu.sync_copy(zeros_hbm.at[r], z_sh.at[r]))
    _over_rows(s, lambda r: pltpu.sync_copy(y_hbm.at[r], y_sh.at[r]))

    pltpu.sync_copy(src_hbm.at[wid], src_v)
    pltpu.sync_copy(dst_hbm.at[wid], dst_v)
    plsc.subcore_barrier()

    def start_gather(b, j):
      from_hbm = (j % 2) < 1

      @pl.when(from_hbm)
      def _():
        pltpu.async_copy(y_hbm.at[src_v.at[j]], rows_v.at[b], sem.at[b])

      @pl.when(jnp.logical_not(from_hbm))
      def _():
        pltpu.async_copy(y_sh.at[src_v.at[j]], rows_v.at[b], sem.at[b])

    for b in range(NBUF):
      start_gather(b, b)

    def outer(o, _):
      for b in range(NBUF):
        j = o * NBUF + b
        pltpu.make_async_copy(
            y_sh.at[src_v.at[j]], rows_v.at[b], sem.at[b]).wait()
        pltpu.sync_copy(rows_v.at[b], z_sh.at[dst_v.at[j]], add=True)
        nxt = j + NBUF

        @pl.when(nxt < CHUNKS)
        def _():
          start_gather(b, nxt)
      return 0

    lax.fori_loop(0, NOUT, outer, 0)
    plsc.subcore_barrier()

    @pl.when(c == 0)
    def _():
      _over_rows(s, lambda r: pltpu.sync_copy(z_sh.at[r], out0_hbm.at[r]))

    @pl.when(c != 0)
    def _():
      _over_rows(s, lambda r: pltpu.sync_copy(z_sh.at[r], out1_hbm.at[r]))

  return pl.kernel(
      body,
      out_type=[
          jax.ShapeDtypeStruct((N, DIM), jnp.float32),
          jax.ShapeDtypeStruct((N, DIM), jnp.float32),
      ],
      mesh=mesh,
      scratch_types=scratch,
      compiler_params=pltpu.CompilerParams(use_tc_tiling_on_sc=False),
  )


P = 8
NB = N // P
LANES = P * DIM


def _blockdiag(w):
  k, m = w.shape
  return (jnp.eye(P, dtype=w.dtype)[:, None, :, None]
          * w[None, :, None, :]).reshape(P * k, P * m)


def _tc_mm1_body(x8_ref, w_ref, xt_ref):
  xt_ref[...] = jnp.dot(x8_ref[...], w_ref[...],
                        preferred_element_type=jnp.float32)


_tc_mm1 = pl.pallas_call(
    _tc_mm1_body,
    out_shape=jax.ShapeDtypeStruct((NB, LANES), jnp.float32),
)


def _tc_prep1_body(d0_ref, d1_ref, xt_ref, y_ref, dinv_ref):
  deg = d0_ref[...] + d1_ref[...] + 1.0
  dinv = lax.rsqrt(deg)
  dinv_ref[...] = dinv
  y_ref[...] = dinv * xt_ref[...]


_tc_prep1 = pl.pallas_call(
    _tc_prep1_body,
    out_shape=[
        jax.ShapeDtypeStruct((NB, LANES), jnp.float32),
        jax.ShapeDtypeStruct((NB, LANES), jnp.float32),
    ],
)


def _tc_mid_body(z0_ref, z1_ref, yin_ref, dinv_ref, b_ref, w_ref, y_ref):
  dinv = dinv_ref[...]
  h = jax.nn.relu(dinv * (z0_ref[...] + z1_ref[...] + yin_ref[...])
                  + b_ref[...])
  y_ref[...] = dinv * jnp.dot(h, w_ref[...],
                              preferred_element_type=jnp.float32)


_tc_mid = pl.pallas_call(
    _tc_mid_body,
    out_shape=jax.ShapeDtypeStruct((NB, LANES), jnp.float32),
)


def _tc_final_body(z0_ref, z1_ref, yin_ref, dinv_ref, b_ref, wl_ref, bl_ref,
                   g_ref, o_ref):
  h = dinv_ref[...] * (z0_ref[...] + z1_ref[...] + yin_ref[...]) + b_ref[...]
  lg = jnp.dot(h, wl_ref[...], preferred_element_type=jnp.float32)
  lg = lg + bl_ref[...]
  m = jnp.max(lg, axis=1, keepdims=True)
  ex = jnp.exp(lg - m)
  s = jnp.dot(ex, g_ref[...], preferred_element_type=jnp.float32)
  o_ref[...] = (lg - m) - jnp.log(s)


_tc_final = pl.pallas_call(
    _tc_final_body,
    out_shape=jax.ShapeDtypeStruct((NB, LANES), jnp.float32),
)


def kernel(x, edge_index, W1, b1, W2, b2, W3, b3, Wl, bl):
  src_p = edge_index[0].reshape(NT, CHUNKS, CH)
  dst_p = edge_index[1].reshape(NT, CHUNKS, CH)
  dst_f = edge_index[1].reshape(NT, CHUNKS * CH)

  zeros = jnp.zeros((N, DIM), jnp.float32)
  zeros1 = jnp.zeros((N,), jnp.float32)

  sc_layer = _make_sc_agg()
  sc_deg = _make_sc_deg()

  x8 = x.reshape(NB, P * D_IN)
  w1b = _blockdiag(W1)
  w2b = _blockdiag(W2)
  w3b = _blockdiag(W3)
  wlb = _blockdiag(Wl)
  gmat = _blockdiag(jnp.ones((DIM, N_CLASSES), jnp.float32))
  b1p = jnp.tile(b1, P).reshape(1, LANES)
  b2p = jnp.tile(b2, P).reshape(1, LANES)
  b3p = jnp.tile(b3, P).reshape(1, LANES)
  blp = jnp.tile(bl, P).reshape(1, LANES)

  pk = lambda a: a.reshape(NB, LANES)

  xt1 = _tc_mm1(x8, w1b)
  d0, d1 = sc_deg(dst_f, zeros1)
  y1, dinv = _tc_prep1(pk(d0), pk(d1), xt1)
  z0, z1 = sc_layer(y1.reshape(N, DIM), src_p, dst_p, zeros)
  y2 = _tc_mid(pk(z0), pk(z1), y1, dinv, b1p, w2b)
  z0, z1 = sc_layer(y2.reshape(N, DIM), src_p, dst_p, zeros)
  y3 = _tc_mid(pk(z0), pk(z1), y2, dinv, b2p, w3b)
  z0, z1 = sc_layer(y3.reshape(N, DIM), src_p, dst_p, zeros)
  out = _tc_final(pk(z0), pk(z1), y3, dinv, b3p, wlb, blp, gmat)
  return out.reshape(N, N_CLASSES)


# speedup vs baseline: 1.0629x; 1.0159x over previous
import functools

import jax
import jax.numpy as jnp
from jax import lax
from jax.experimental import pallas as pl
from jax.experimental.pallas import tpu as pltpu
from jax.experimental.pallas import tpu_sc as plsc

N = 10000
D_IN = 128
DIM = 16
N_CLASSES = 16
E = 320000

NC = 2
NS = 16
NT = NC * NS
CH = 125
CHUNKS = 80
NBUF = 8
NOUT = CHUNKS // NBUF
NP = N
RPT = 632
RPT_LAST = N - (NS - 1) * RPT


def _over_rows(s, fn):
  @pl.when(s < NS - 1)
  def _():
    fn(pl.ds(s * RPT, RPT))

  @pl.when(s == NS - 1)
  def _():
    fn(pl.ds((NS - 1) * RPT, RPT_LAST))

RPT16 = 640
RPT16_LAST = N - (NS - 1) * RPT16


@functools.lru_cache(maxsize=None)
def _make_sc_deg():
  mesh = plsc.VectorSubcoreMesh(core_axis_name="c", subcore_axis_name="s",
                                num_cores=NC, num_subcores=NS)
  EPT = CHUNKS * CH

  scratch = [
      pltpu.VMEM((EPT,), jnp.int32),
      pltpu.VMEM((N,), jnp.float32),
      pltpu.VMEM((NS, RPT16), jnp.float32),
      pltpu.VMEM((RPT16, DIM), jnp.float32),
      pltpu.VMEM_SHARED((NS, N), jnp.float32),
  ]

  def body(dstf_hbm, zeros1_hbm, out0_hbm, out1_hbm,
           dst_f, hist, tmp_v, rep_v, hist_sh):
    c = lax.axis_index("c")
    s = lax.axis_index("s")
    wid = s * NC + c
    pltpu.sync_copy(dstf_hbm.at[wid], dst_f)
    pltpu.sync_copy(zeros1_hbm, hist)
    ones16 = jnp.full((16,), 1.0, jnp.float32)

    def step(i, _):
      idx = dst_f[pl.ds(i * 16, 16)]
      plsc.addupdate_scatter(hist, [idx], ones16)
      return 0

    lax.fori_loop(0, EPT // 16, step, 0)
    pltpu.sync_copy(hist, hist_sh.at[s])
    plsc.subcore_barrier()

    def reduce_range(base, length):
      for t in range(NS):
        pltpu.sync_copy(hist_sh.at[t, pl.ds(base, length)],
                        tmp_v.at[t, pl.ds(0, length)])

      def rstep(b2, _):
        acc = tmp_v[0, pl.ds(b2 * 16, 16)]
        for t in range(1, NS):
          acc = acc + tmp_v[t, pl.ds(b2 * 16, 16)]
        for k in range(16):
          rep_v[b2 * 16 + k, :] = jnp.full((DIM,), acc[k], jnp.float32)
        return 0

      lax.fori_loop(0, length // 16, rstep, 0)

      @pl.when(c == 0)
      def _():
        pltpu.sync_copy(rep_v.at[pl.ds(0, length)],
                        out0_hbm.at[pl.ds(base, length)])

      @pl.when(c != 0)
      def _():
        pltpu.sync_copy(rep_v.at[pl.ds(0, length)],
                        out1_hbm.at[pl.ds(base, length)])

    @pl.when(s < NS - 1)
    def _():
      reduce_range(s * RPT16, RPT16)

    @pl.when(s == NS - 1)
    def _():
      reduce_range((NS - 1) * RPT16, RPT16_LAST)

  return pl.kernel(
      body,
      out_type=[
          jax.ShapeDtypeStruct((N, DIM), jnp.float32),
          jax.ShapeDtypeStruct((N, DIM), jnp.float32),
      ],
      mesh=mesh,
      scratch_types=scratch,
      compiler_params=pltpu.CompilerParams(use_tc_tiling_on_sc=False,
                                           needs_layout_passes=False),
  )


@functools.lru_cache(maxsize=None)
def _make_sc_agg():
  mesh = plsc.VectorSubcoreMesh(core_axis_name="c", subcore_axis_name="s",
                                num_cores=NC, num_subcores=NS)

  scratch = [
      pltpu.VMEM((CHUNKS, CH), jnp.int32),
      pltpu.VMEM((CHUNKS, CH), jnp.int32),
      pltpu.VMEM((NBUF, CH, DIM), jnp.float32),
      pltpu.VMEM_SHARED((NP, DIM), jnp.float32),
      pltpu.VMEM_SHARED((N, DIM), jnp.float32),
      pltpu.SemaphoreType.DMA((NBUF,)),
  ]

  def body(y_hbm, src_hbm, dst_hbm, zeros_hbm, out0_hbm, out1_hbm,
           src_v, dst_v, rows_v, z_sh, y_sh, sem):
    c = lax.axis_index("c")
    s = lax.axis_index("s")
    wid = s * NC + c

    _over_rows(s, lambda r: pltpu.sync_copy(zeros_hbm.at[r], z_sh.at[r]))
    _over_rows(s, lambda r: pltpu.sync_copy(y_hbm.at[r], y_sh.at[r]))

    pltpu.sync_copy(src_hbm.at[wid], src_v)
    pltpu.sync_copy(dst_hbm.at[wid], dst_v)
    plsc.subcore_barrier()

    def start_gather(b, j):
      from_hbm = (j % 3) < 2

      @pl.when(from_hbm)
      def _():
        pltpu.async_copy(y_hbm.at[src_v.at[j]], rows_v.at[b], sem.at[b])

      @pl.when(jnp.logical_not(from_hbm))
      def _():
        pltpu.async_copy(y_sh.at[src_v.at[j]], rows_v.at[b], sem.at[b])

    for b in range(NBUF):
      start_gather(b, b)

    def outer(o, _):
      for b in range(NBUF):
        j = o * NBUF + b
        pltpu.make_async_copy(
            y_sh.at[src_v.at[j]], rows_v.at[b], sem.at[b]).wait()
        pltpu.sync_copy(rows_v.at[b], z_sh.at[dst_v.at[j]], add=True)
        nxt = j + NBUF

        @pl.when(nxt < CHUNKS)
        def _():
          start_gather(b, nxt)
      return 0

    lax.fori_loop(0, NOUT, outer, 0)
    plsc.subcore_barrier()

    @pl.when(c == 0)
    def _():
      _over_rows(s, lambda r: pltpu.sync_copy(z_sh.at[r], out0_hbm.at[r]))

    @pl.when(c != 0)
    def _():
      _over_rows(s, lambda r: pltpu.sync_copy(z_sh.at[r], out1_hbm.at[r]))

  return pl.kernel(
      body,
      out_type=[
          jax.ShapeDtypeStruct((N, DIM), jnp.float32),
          jax.ShapeDtypeStruct((N, DIM), jnp.float32),
      ],
      mesh=mesh,
      scratch_types=scratch,
      compiler_params=pltpu.CompilerParams(use_tc_tiling_on_sc=False),
  )


P = 8
NB = N // P
LANES = P * DIM


def _blockdiag(w):
  k, m = w.shape
  return (jnp.eye(P, dtype=w.dtype)[:, None, :, None]
          * w[None, :, None, :]).reshape(P * k, P * m)


def _tc_mm1_body(x8_ref, w_ref, xt_ref):
  xt_ref[...] = jnp.dot(x8_ref[...], w_ref[...],
                        preferred_element_type=jnp.float32)


_tc_mm1 = pl.pallas_call(
    _tc_mm1_body,
    out_shape=jax.ShapeDtypeStruct((NB, LANES), jnp.float32),
)


def _tc_prep1_body(d0_ref, d1_ref, xt_ref, y_ref, dinv_ref):
  deg = d0_ref[...] + d1_ref[...] + 1.0
  dinv = lax.rsqrt(deg)
  dinv_ref[...] = dinv
  y_ref[...] = dinv * xt_ref[...]


_tc_prep1 = pl.pallas_call(
    _tc_prep1_body,
    out_shape=[
        jax.ShapeDtypeStruct((NB, LANES), jnp.float32),
        jax.ShapeDtypeStruct((NB, LANES), jnp.float32),
    ],
)


def _tc_mid_body(z0_ref, z1_ref, yin_ref, dinv_ref, b_ref, w_ref, y_ref):
  dinv = dinv_ref[...]
  h = jax.nn.relu(dinv * (z0_ref[...] + z1_ref[...] + yin_ref[...])
                  + b_ref[...])
  y_ref[...] = dinv * jnp.dot(h, w_ref[...],
                              preferred_element_type=jnp.float32)


_tc_mid = pl.pallas_call(
    _tc_mid_body,
    out_shape=jax.ShapeDtypeStruct((NB, LANES), jnp.float32),
)


def _tc_final_body(z0_ref, z1_ref, yin_ref, dinv_ref, b_ref, wl_ref, bl_ref,
                   g_ref, o_ref):
  h = dinv_ref[...] * (z0_ref[...] + z1_ref[...] + yin_ref[...]) + b_ref[...]
  lg = jnp.dot(h, wl_ref[...], preferred_element_type=jnp.float32)
  lg = lg + bl_ref[...]
  m = jnp.max(lg, axis=1, keepdims=True)
  ex = jnp.exp(lg - m)
  s = jnp.dot(ex, g_ref[...], preferred_element_type=jnp.float32)
  o_ref[...] = (lg - m) - jnp.log(s)


_tc_final = pl.pallas_call(
    _tc_final_body,
    out_shape=jax.ShapeDtypeStruct((NB, LANES), jnp.float32),
)


def kernel(x, edge_index, W1, b1, W2, b2, W3, b3, Wl, bl):
  src_p = edge_index[0].reshape(NT, CHUNKS, CH)
  dst_p = edge_index[1].reshape(NT, CHUNKS, CH)
  dst_f = edge_index[1].reshape(NT, CHUNKS * CH)

  zeros = jnp.zeros((N, DIM), jnp.float32)
  zeros1 = jnp.zeros((N,), jnp.float32)

  sc_layer = _make_sc_agg()
  sc_deg = _make_sc_deg()

  x8 = x.reshape(NB, P * D_IN)
  w1b = _blockdiag(W1)
  w2b = _blockdiag(W2)
  w3b = _blockdiag(W3)
  wlb = _blockdiag(Wl)
  gmat = _blockdiag(jnp.ones((DIM, N_CLASSES), jnp.float32))
  b1p = jnp.tile(b1, P).reshape(1, LANES)
  b2p = jnp.tile(b2, P).reshape(1, LANES)
  b3p = jnp.tile(b3, P).reshape(1, LANES)
  blp = jnp.tile(bl, P).reshape(1, LANES)

  pk = lambda a: a.reshape(NB, LANES)

  xt1 = _tc_mm1(x8, w1b)
  d0, d1 = sc_deg(dst_f, zeros1)
  y1, dinv = _tc_prep1(pk(d0), pk(d1), xt1)
  z0, z1 = sc_layer(y1.reshape(N, DIM), src_p, dst_p, zeros)
  y2 = _tc_mid(pk(z0), pk(z1), y1, dinv, b1p, w2b)
  z0, z1 = sc_layer(y2.reshape(N, DIM), src_p, dst_p, zeros)
  y3 = _tc_mid(pk(z0), pk(z1), y2, dinv, b2p, w3b)
  z0, z1 = sc_layer(y3.reshape(N, DIM), src_p, dst_p, zeros)
  out = _tc_final(pk(z0), pk(z1), y3, dinv, b3p, wlb, blp, gmat)
  return out.reshape(N, N_CLASSES)


# speedup vs baseline: 1.0774x; 1.0137x over previous
import functools

import jax
import jax.numpy as jnp
from jax import lax
from jax.experimental import pallas as pl
from jax.experimental.pallas import tpu as pltpu
from jax.experimental.pallas import tpu_sc as plsc

N = 10000
D_IN = 128
DIM = 16
N_CLASSES = 16
E = 320000

NC = 2
NS = 16
NT = NC * NS
CH = 125
CHUNKS = 80
NBUF = 8
NOUT = CHUNKS // NBUF
NP = N
RPT = 632
RPT_LAST = N - (NS - 1) * RPT


def _over_rows(s, fn):
  @pl.when(s < NS - 1)
  def _():
    fn(pl.ds(s * RPT, RPT))

  @pl.when(s == NS - 1)
  def _():
    fn(pl.ds((NS - 1) * RPT, RPT_LAST))

RPT16 = 640
RPT16_LAST = N - (NS - 1) * RPT16


@functools.lru_cache(maxsize=None)
def _make_sc_deg():
  mesh = plsc.VectorSubcoreMesh(core_axis_name="c", subcore_axis_name="s",
                                num_cores=NC, num_subcores=NS)
  EPT = CHUNKS * CH

  scratch = [
      pltpu.VMEM((EPT,), jnp.int32),
      pltpu.VMEM((N,), jnp.float32),
      pltpu.VMEM((NS, RPT16), jnp.float32),
      pltpu.VMEM((RPT16, DIM), jnp.float32),
      pltpu.VMEM_SHARED((NS, N), jnp.float32),
  ]

  def body(dstf_hbm, zeros1_hbm, out0_hbm, out1_hbm,
           dst_f, hist, tmp_v, rep_v, hist_sh):
    c = lax.axis_index("c")
    s = lax.axis_index("s")
    wid = s * NC + c
    pltpu.sync_copy(dstf_hbm.at[wid], dst_f)
    pltpu.sync_copy(zeros1_hbm, hist)
    ones16 = jnp.full((16,), 1.0, jnp.float32)

    def step(i, _):
      idx = dst_f[pl.ds(i * 16, 16)]
      plsc.addupdate_scatter(hist, [idx], ones16)
      return 0

    lax.fori_loop(0, EPT // 16, step, 0)
    pltpu.sync_copy(hist, hist_sh.at[s])
    plsc.subcore_barrier()

    def reduce_range(base, length):
      for t in range(NS):
        pltpu.sync_copy(hist_sh.at[t, pl.ds(base, length)],
                        tmp_v.at[t, pl.ds(0, length)])

      def rstep(b2, _):
        acc = tmp_v[0, pl.ds(b2 * 16, 16)]
        for t in range(1, NS):
          acc = acc + tmp_v[t, pl.ds(b2 * 16, 16)]
        for k in range(16):
          rep_v[b2 * 16 + k, :] = jnp.full((DIM,), acc[k], jnp.float32)
        return 0

      lax.fori_loop(0, length // 16, rstep, 0)

      @pl.when(c == 0)
      def _():
        pltpu.sync_copy(rep_v.at[pl.ds(0, length)],
                        out0_hbm.at[pl.ds(base, length)])

      @pl.when(c != 0)
      def _():
        pltpu.sync_copy(rep_v.at[pl.ds(0, length)],
                        out1_hbm.at[pl.ds(base, length)])

    @pl.when(s < NS - 1)
    def _():
      reduce_range(s * RPT16, RPT16)

    @pl.when(s == NS - 1)
    def _():
      reduce_range((NS - 1) * RPT16, RPT16_LAST)

  return pl.kernel(
      body,
      out_type=[
          jax.ShapeDtypeStruct((N, DIM), jnp.float32),
          jax.ShapeDtypeStruct((N, DIM), jnp.float32),
      ],
      mesh=mesh,
      scratch_types=scratch,
      compiler_params=pltpu.CompilerParams(use_tc_tiling_on_sc=False,
                                           needs_layout_passes=False),
  )


@functools.lru_cache(maxsize=None)
def _make_sc_agg():
  mesh = plsc.VectorSubcoreMesh(core_axis_name="c", subcore_axis_name="s",
                                num_cores=NC, num_subcores=NS)

  scratch = [
      pltpu.VMEM((CHUNKS, CH), jnp.int32),
      pltpu.VMEM((CHUNKS, CH), jnp.int32),
      pltpu.VMEM((NBUF, CH, DIM), jnp.float32),
      pltpu.VMEM_SHARED((NP, DIM), jnp.float32),
      pltpu.VMEM_SHARED((N, DIM), jnp.float32),
      pltpu.SemaphoreType.DMA((NBUF,)),
  ]

  def body(y_hbm, src_hbm, dst_hbm, zeros_hbm, out0_hbm, out1_hbm,
           src_v, dst_v, rows_v, z_sh, y_sh, sem):
    c = lax.axis_index("c")
    s = lax.axis_index("s")
    wid = s * NC + c

    _over_rows(s, lambda r: pltpu.sync_copy(zeros_hbm.at[r], z_sh.at[r]))
    _over_rows(s, lambda r: pltpu.sync_copy(y_hbm.at[r], y_sh.at[r]))

    pltpu.sync_copy(src_hbm.at[wid], src_v)
    pltpu.sync_copy(dst_hbm.at[wid], dst_v)
    plsc.subcore_barrier()

    def start_gather(b, j):
      from_hbm = (j % 4) < 3

      @pl.when(from_hbm)
      def _():
        pltpu.async_copy(y_hbm.at[src_v.at[j]], rows_v.at[b], sem.at[b])

      @pl.when(jnp.logical_not(from_hbm))
      def _():
        pltpu.async_copy(y_sh.at[src_v.at[j]], rows_v.at[b], sem.at[b])

    for b in range(NBUF):
      start_gather(b, b)

    def outer(o, _):
      for b in range(NBUF):
        j = o * NBUF + b
        pltpu.make_async_copy(
            y_sh.at[src_v.at[j]], rows_v.at[b], sem.at[b]).wait()
        pltpu.sync_copy(rows_v.at[b], z_sh.at[dst_v.at[j]], add=True)
        nxt = j + NBUF

        @pl.when(nxt < CHUNKS)
        def _():
          start_gather(b, nxt)
      return 0

    lax.fori_loop(0, NOUT, outer, 0)
    plsc.subcore_barrier()

    @pl.when(c == 0)
    def _():
      _over_rows(s, lambda r: pltpu.sync_copy(z_sh.at[r], out0_hbm.at[r]))

    @pl.when(c != 0)
    def _():
      _over_rows(s, lambda r: pltpu.sync_copy(z_sh.at[r], out1_hbm.at[r]))

  return pl.kernel(
      body,
      out_type=[
          jax.ShapeDtypeStruct((N, DIM), jnp.float32),
          jax.ShapeDtypeStruct((N, DIM), jnp.float32),
      ],
      mesh=mesh,
      scratch_types=scratch,
      compiler_params=pltpu.CompilerParams(use_tc_tiling_on_sc=False),
  )


P = 8
NB = N // P
LANES = P * DIM


def _blockdiag(w):
  k, m = w.shape
  return (jnp.eye(P, dtype=w.dtype)[:, None, :, None]
          * w[None, :, None, :]).reshape(P * k, P * m)


def _tc_mm1_body(x8_ref, w_ref, xt_ref):
  xt_ref[...] = jnp.dot(x8_ref[...], w_ref[...],
                        preferred_element_type=jnp.float32)


_tc_mm1 = pl.pallas_call(
    _tc_mm1_body,
    out_shape=jax.ShapeDtypeStruct((NB, LANES), jnp.float32),
)


def _tc_prep1_body(d0_ref, d1_ref, xt_ref, y_ref, dinv_ref):
  deg = d0_ref[...] + d1_ref[...] + 1.0
  dinv = lax.rsqrt(deg)
  dinv_ref[...] = dinv
  y_ref[...] = dinv * xt_ref[...]


_tc_prep1 = pl.pallas_call(
    _tc_prep1_body,
    out_shape=[
        jax.ShapeDtypeStruct((NB, LANES), jnp.float32),
        jax.ShapeDtypeStruct((NB, LANES), jnp.float32),
    ],
)


def _tc_mid_body(z0_ref, z1_ref, yin_ref, dinv_ref, b_ref, w_ref, y_ref):
  dinv = dinv_ref[...]
  h = jax.nn.relu(dinv * (z0_ref[...] + z1_ref[...] + yin_ref[...])
                  + b_ref[...])
  y_ref[...] = dinv * jnp.dot(h, w_ref[...],
                              preferred_element_type=jnp.float32)


_tc_mid = pl.pallas_call(
    _tc_mid_body,
    out_shape=jax.ShapeDtypeStruct((NB, LANES), jnp.float32),
)


def _tc_final_body(z0_ref, z1_ref, yin_ref, dinv_ref, b_ref, wl_ref, bl_ref,
                   g_ref, o_ref):
  h = dinv_ref[...] * (z0_ref[...] + z1_ref[...] + yin_ref[...]) + b_ref[...]
  lg = jnp.dot(h, wl_ref[...], preferred_element_type=jnp.float32)
  lg = lg + bl_ref[...]
  m = jnp.max(lg, axis=1, keepdims=True)
  ex = jnp.exp(lg - m)
  s = jnp.dot(ex, g_ref[...], preferred_element_type=jnp.float32)
  o_ref[...] = (lg - m) - jnp.log(s)


_tc_final = pl.pallas_call(
    _tc_final_body,
    out_shape=jax.ShapeDtypeStruct((NB, LANES), jnp.float32),
)


def kernel(x, edge_index, W1, b1, W2, b2, W3, b3, Wl, bl):
  src_p = edge_index[0].reshape(NT, CHUNKS, CH)
  dst_p = edge_index[1].reshape(NT, CHUNKS, CH)
  dst_f = edge_index[1].reshape(NT, CHUNKS * CH)

  zeros = jnp.zeros((N, DIM), jnp.float32)
  zeros1 = jnp.zeros((N,), jnp.float32)

  sc_layer = _make_sc_agg()
  sc_deg = _make_sc_deg()

  x8 = x.reshape(NB, P * D_IN)
  w1b = _blockdiag(W1)
  w2b = _blockdiag(W2)
  w3b = _blockdiag(W3)
  wlb = _blockdiag(Wl)
  gmat = _blockdiag(jnp.ones((DIM, N_CLASSES), jnp.float32))
  b1p = jnp.tile(b1, P).reshape(1, LANES)
  b2p = jnp.tile(b2, P).reshape(1, LANES)
  b3p = jnp.tile(b3, P).reshape(1, LANES)
  blp = jnp.tile(bl, P).reshape(1, LANES)

  pk = lambda a: a.reshape(NB, LANES)

  xt1 = _tc_mm1(x8, w1b)
  d0, d1 = sc_deg(dst_f, zeros1)
  y1, dinv = _tc_prep1(pk(d0), pk(d1), xt1)
  z0, z1 = sc_layer(y1.reshape(N, DIM), src_p, dst_p, zeros)
  y2 = _tc_mid(pk(z0), pk(z1), y1, dinv, b1p, w2b)
  z0, z1 = sc_layer(y2.reshape(N, DIM), src_p, dst_p, zeros)
  y3 = _tc_mid(pk(z0), pk(z1), y2, dinv, b2p, w3b)
  z0, z1 = sc_layer(y3.reshape(N, DIM), src_p, dst_p, zeros)
  out = _tc_final(pk(z0), pk(z1), y3, dinv, b3p, wlb, blp, gmat)
  return out.reshape(N, N_CLASSES)


# speedup vs baseline: 1.0826x; 1.0048x over previous
import functools

import jax
import jax.numpy as jnp
from jax import lax
from jax.experimental import pallas as pl
from jax.experimental.pallas import tpu as pltpu
from jax.experimental.pallas import tpu_sc as plsc

N = 10000
D_IN = 128
DIM = 16
N_CLASSES = 16
E = 320000

NC = 2
NS = 16
NT = NC * NS
CH = 125
CHUNKS = 80
NBUF = 8
NOUT = CHUNKS // NBUF
NP = N
RPT = 632
RPT_LAST = N - (NS - 1) * RPT


def _over_rows(s, fn):
  @pl.when(s < NS - 1)
  def _():
    fn(pl.ds(s * RPT, RPT))

  @pl.when(s == NS - 1)
  def _():
    fn(pl.ds((NS - 1) * RPT, RPT_LAST))

RPT16 = 640
RPT16_LAST = N - (NS - 1) * RPT16


@functools.lru_cache(maxsize=None)
def _make_sc_deg():
  mesh = plsc.VectorSubcoreMesh(core_axis_name="c", subcore_axis_name="s",
                                num_cores=NC, num_subcores=NS)
  EPT = CHUNKS * CH

  scratch = [
      pltpu.VMEM((EPT,), jnp.int32),
      pltpu.VMEM((N,), jnp.float32),
      pltpu.VMEM((NS, RPT16), jnp.float32),
      pltpu.VMEM((RPT16, DIM), jnp.float32),
      pltpu.VMEM_SHARED((NS, N), jnp.float32),
  ]

  def body(dstf_hbm, zeros1_hbm, out0_hbm, out1_hbm,
           dst_f, hist, tmp_v, rep_v, hist_sh):
    c = lax.axis_index("c")
    s = lax.axis_index("s")
    wid = s * NC + c
    pltpu.sync_copy(dstf_hbm.at[wid], dst_f)
    pltpu.sync_copy(zeros1_hbm, hist)
    ones16 = jnp.full((16,), 1.0, jnp.float32)

    def step(i, _):
      idx = dst_f[pl.ds(i * 16, 16)]
      plsc.addupdate_scatter(hist, [idx], ones16)
      return 0

    lax.fori_loop(0, EPT // 16, step, 0)
    pltpu.sync_copy(hist, hist_sh.at[s])
    plsc.subcore_barrier()

    def reduce_range(base, length):
      for t in range(NS):
        pltpu.sync_copy(hist_sh.at[t, pl.ds(base, length)],
                        tmp_v.at[t, pl.ds(0, length)])

      def rstep(b2, _):
        acc = tmp_v[0, pl.ds(b2 * 16, 16)]
        for t in range(1, NS):
          acc = acc + tmp_v[t, pl.ds(b2 * 16, 16)]
        for k in range(16):
          rep_v[b2 * 16 + k, :] = jnp.full((DIM,), acc[k], jnp.float32)
        return 0

      lax.fori_loop(0, length // 16, rstep, 0)

      @pl.when(c == 0)
      def _():
        pltpu.sync_copy(rep_v.at[pl.ds(0, length)],
                        out0_hbm.at[pl.ds(base, length)])

      @pl.when(c != 0)
      def _():
        pltpu.sync_copy(rep_v.at[pl.ds(0, length)],
                        out1_hbm.at[pl.ds(base, length)])

    @pl.when(s < NS - 1)
    def _():
      reduce_range(s * RPT16, RPT16)

    @pl.when(s == NS - 1)
    def _():
      reduce_range((NS - 1) * RPT16, RPT16_LAST)

  return pl.kernel(
      body,
      out_type=[
          jax.ShapeDtypeStruct((N, DIM), jnp.float32),
          jax.ShapeDtypeStruct((N, DIM), jnp.float32),
      ],
      mesh=mesh,
      scratch_types=scratch,
      compiler_params=pltpu.CompilerParams(use_tc_tiling_on_sc=False,
                                           needs_layout_passes=False),
  )


@functools.lru_cache(maxsize=None)
def _make_sc_agg():
  mesh = plsc.VectorSubcoreMesh(core_axis_name="c", subcore_axis_name="s",
                                num_cores=NC, num_subcores=NS)

  scratch = [
      pltpu.VMEM((CHUNKS, CH), jnp.int32),
      pltpu.VMEM((CHUNKS, CH), jnp.int32),
      pltpu.VMEM((NBUF, CH, DIM), jnp.float32),
      pltpu.VMEM_SHARED((NP, DIM), jnp.float32),
      pltpu.VMEM_SHARED((N, DIM), jnp.float32),
      pltpu.SemaphoreType.DMA((NBUF,)),
  ]

  def body(y_hbm, src_hbm, dst_hbm, zeros_hbm, out0_hbm, out1_hbm,
           src_v, dst_v, rows_v, z_sh, y_sh, sem):
    c = lax.axis_index("c")
    s = lax.axis_index("s")
    wid = s * NC + c

    _over_rows(s, lambda r: pltpu.sync_copy(zeros_hbm.at[r], z_sh.at[r]))
    _over_rows(s, lambda r: pltpu.sync_copy(y_hbm.at[r], y_sh.at[r]))

    pltpu.sync_copy(src_hbm.at[wid], src_v)
    pltpu.sync_copy(dst_hbm.at[wid], dst_v)
    plsc.subcore_barrier()

    def start_gather(b, j):
      from_hbm = (j % 8) < 7

      @pl.when(from_hbm)
      def _():
        pltpu.async_copy(y_hbm.at[src_v.at[j]], rows_v.at[b], sem.at[b])

      @pl.when(jnp.logical_not(from_hbm))
      def _():
        pltpu.async_copy(y_sh.at[src_v.at[j]], rows_v.at[b], sem.at[b])

    for b in range(NBUF):
      start_gather(b, b)

    def outer(o, _):
      for b in range(NBUF):
        j = o * NBUF + b
        pltpu.make_async_copy(
            y_sh.at[src_v.at[j]], rows_v.at[b], sem.at[b]).wait()
        pltpu.sync_copy(rows_v.at[b], z_sh.at[dst_v.at[j]], add=True)
        nxt = j + NBUF

        @pl.when(nxt < CHUNKS)
        def _():
          start_gather(b, nxt)
      return 0

    lax.fori_loop(0, NOUT, outer, 0)
    plsc.subcore_barrier()

    @pl.when(c == 0)
    def _():
      _over_rows(s, lambda r: pltpu.sync_copy(z_sh.at[r], out0_hbm.at[r]))

    @pl.when(c != 0)
    def _():
      _over_rows(s, lambda r: pltpu.sync_copy(z_sh.at[r], out1_hbm.at[r]))

  return pl.kernel(
      body,
      out_type=[
          jax.ShapeDtypeStruct((N, DIM), jnp.float32),
          jax.ShapeDtypeStruct((N, DIM), jnp.float32),
      ],
      mesh=mesh,
      scratch_types=scratch,
      compiler_params=pltpu.CompilerParams(use_tc_tiling_on_sc=False),
  )


P = 8
NB = N // P
LANES = P * DIM


def _blockdiag(w):
  k, m = w.shape
  return (jnp.eye(P, dtype=w.dtype)[:, None, :, None]
          * w[None, :, None, :]).reshape(P * k, P * m)


def _tc_mm1_body(x8_ref, w_ref, xt_ref):
  xt_ref[...] = jnp.dot(x8_ref[...], w_ref[...],
                        preferred_element_type=jnp.float32)


_tc_mm1 = pl.pallas_call(
    _tc_mm1_body,
    out_shape=jax.ShapeDtypeStruct((NB, LANES), jnp.float32),
)


def _tc_prep1_body(d0_ref, d1_ref, xt_ref, y_ref, dinv_ref):
  deg = d0_ref[...] + d1_ref[...] + 1.0
  dinv = lax.rsqrt(deg)
  dinv_ref[...] = dinv
  y_ref[...] = dinv * xt_ref[...]


_tc_prep1 = pl.pallas_call(
    _tc_prep1_body,
    out_shape=[
        jax.ShapeDtypeStruct((NB, LANES), jnp.float32),
        jax.ShapeDtypeStruct((NB, LANES), jnp.float32),
    ],
)


def _tc_mid_body(z0_ref, z1_ref, yin_ref, dinv_ref, b_ref, w_ref, y_ref):
  dinv = dinv_ref[...]
  h = jax.nn.relu(dinv * (z0_ref[...] + z1_ref[...] + yin_ref[...])
                  + b_ref[...])
  y_ref[...] = dinv * jnp.dot(h, w_ref[...],
                              preferred_element_type=jnp.float32)


_tc_mid = pl.pallas_call(
    _tc_mid_body,
    out_shape=jax.ShapeDtypeStruct((NB, LANES), jnp.float32),
)


def _tc_final_body(z0_ref, z1_ref, yin_ref, dinv_ref, b_ref, wl_ref, bl_ref,
                   g_ref, o_ref):
  h = dinv_ref[...] * (z0_ref[...] + z1_ref[...] + yin_ref[...]) + b_ref[...]
  lg = jnp.dot(h, wl_ref[...], preferred_element_type=jnp.float32)
  lg = lg + bl_ref[...]
  m = jnp.max(lg, axis=1, keepdims=True)
  ex = jnp.exp(lg - m)
  s = jnp.dot(ex, g_ref[...], preferred_element_type=jnp.float32)
  o_ref[...] = (lg - m) - jnp.log(s)


_tc_final = pl.pallas_call(
    _tc_final_body,
    out_shape=jax.ShapeDtypeStruct((NB, LANES), jnp.float32),
)


def kernel(x, edge_index, W1, b1, W2, b2, W3, b3, Wl, bl):
  src_p = edge_index[0].reshape(NT, CHUNKS, CH)
  dst_p = edge_index[1].reshape(NT, CHUNKS, CH)
  dst_f = edge_index[1].reshape(NT, CHUNKS * CH)

  zeros = jnp.zeros((N, DIM), jnp.float32)
  zeros1 = jnp.zeros((N,), jnp.float32)

  sc_layer = _make_sc_agg()
  sc_deg = _make_sc_deg()

  x8 = x.reshape(NB, P * D_IN)
  w1b = _blockdiag(W1)
  w2b = _blockdiag(W2)
  w3b = _blockdiag(W3)
  wlb = _blockdiag(Wl)
  gmat = _blockdiag(jnp.ones((DIM, N_CLASSES), jnp.float32))
  b1p = jnp.tile(b1, P).reshape(1, LANES)
  b2p = jnp.tile(b2, P).reshape(1, LANES)
  b3p = jnp.tile(b3, P).reshape(1, LANES)
  blp = jnp.tile(bl, P).reshape(1, LANES)

  pk = lambda a: a.reshape(NB, LANES)

  xt1 = _tc_mm1(x8, w1b)
  d0, d1 = sc_deg(dst_f, zeros1)
  y1, dinv = _tc_prep1(pk(d0), pk(d1), xt1)
  z0, z1 = sc_layer(y1.reshape(N, DIM), src_p, dst_p, zeros)
  y2 = _tc_mid(pk(z0), pk(z1), y1, dinv, b1p, w2b)
  z0, z1 = sc_layer(y2.reshape(N, DIM), src_p, dst_p, zeros)
  y3 = _tc_mid(pk(z0), pk(z1), y2, dinv, b2p, w3b)
  z0, z1 = sc_layer(y3.reshape(N, DIM), src_p, dst_p, zeros)
  out = _tc_final(pk(z0), pk(z1), y3, dinv, b3p, wlb, blp, gmat)
  return out.reshape(N, N_CLASSES)
